# Initial kernel scaffold; baseline (speedup 1.0000x reference)
import jax
import jax.numpy as jnp
from jax.experimental import pallas as pl


def kernel(x, edge_index, emb, Wq, bq, Wk, bk, Wv, bv, WO, bO, ln1_g, ln1_b, W1, b1, W2, b2, ln2_g, ln2_b, r0_W, r0_b, r1_W, r1_b, r2_W, r2_b):
    raise NotImplementedError("write your pallas kernel here")


# speedup vs baseline: 31.2857x; 31.2857x over previous
import functools

import jax
import jax.numpy as jnp
from jax import lax
from jax.experimental import pallas as pl
from jax.experimental.pallas import tpu as pltpu
from jax.experimental.pallas import tpu_sc as plsc

N = 10000
E = 160000
D = 128
H = 8
DH = 16
NL = 4
INV_SCALE = 1.0 / 4.0

NW = 32
EB = 96
NB = 53
EPW = EB * NB
E_PAD = NW * EPW
NPAD = 10112
RPT = NPAD // 16
ZF = RPT // EB
ZT = RPT - ZF * EB

BN = 1000
GRID = N // BN


def _full(shape):
    return pl.BlockSpec(shape, lambda i: (0, 0))


def _rows(cols):
    return pl.BlockSpec((BN, cols), lambda i: (i, 0))


def _embed_body(x_ref, emb_ref, o_ref):
    ids = x_ref[...]
    iota = lax.broadcasted_iota(jnp.int32, (BN, 64), 1)
    onehot = (ids == iota).astype(jnp.float32)
    o_ref[...] = jnp.dot(onehot, emb_ref[...], preferred_element_type=jnp.float32)


def _embed(xr, emb):
    return pl.pallas_call(
        _embed_body,
        grid=(GRID,),
        in_specs=[pl.BlockSpec((BN, 1), lambda i: (i, 0)), _full((64, D))],
        out_specs=_rows(D),
        out_shape=jax.ShapeDtypeStruct((N, D), jnp.float32),
    )(xr, emb)


def _qkv_body(h_ref, wq, bq, wk, bk, wv, bv, q_o, k_o, v_o):
    hb = h_ref[...]
    q_o[...] = jnp.dot(hb, wq[...], preferred_element_type=jnp.float32) + bq[...]
    k_o[...] = jnp.dot(hb, wk[...], preferred_element_type=jnp.float32) + bk[...]
    v_o[...] = jnp.dot(hb, wv[...], preferred_element_type=jnp.float32) + bv[...]


def _qkv(h, wq, bq, wk, bk, wv, bv):
    out = jax.ShapeDtypeStruct((N, D), jnp.float32)
    return pl.pallas_call(
        _qkv_body,
        grid=(GRID,),
        in_specs=[_rows(D), _full((D, D)), _full((1, D)), _full((D, D)),
                  _full((1, D)), _full((D, D)), _full((1, D))],
        out_specs=(_rows(D), _rows(D), _rows(D)),
        out_shape=(out, out, out),
    )(h, wq, bq, wk, bk, wv, bv)


def _ln(xv, g, b):
    mu = jnp.mean(xv, axis=-1, keepdims=True)
    var = jnp.mean((xv - mu) ** 2, axis=-1, keepdims=True)
    return (xv - mu) * lax.rsqrt(var + 1e-5) * g + b


def _post_body(m0, m1, zr0, zr1, h_ref, wo, bo, g1, b1n, w1, b1f, w2, b2f, g2, b2n, o_ref):
    wv = m0[0] + m1[0]
    zrep = zr0[0] + zr1[0]
    att = wv / (zrep + 1e-6)
    y = jnp.dot(att, wo[...], preferred_element_type=jnp.float32) + bo[...]
    hb = h_ref[...]
    h1 = _ln(hb + y, g1[...], b1n[...])
    t = jax.nn.relu(jnp.dot(y, w1[...], preferred_element_type=jnp.float32) + b1f[...])
    y2 = jnp.dot(t, w2[...], preferred_element_type=jnp.float32) + b2f[...]
    o_ref[...] = _ln(h1 + y2, g2[...], b2n[...])


def _post(outm, outz, h, wo, bo, g1, b1n, w1, b1f, w2, b2f, g2, b2n):
    part0 = pl.BlockSpec((1, BN, D), lambda i: (0, i, 0))
    part1 = pl.BlockSpec((1, BN, D), lambda i: (1, i, 0))
    return pl.pallas_call(
        _post_body,
        grid=(GRID,),
        in_specs=[part0, part1, part0, part1, _rows(D), _full((D, D)), _full((1, D)),
                  _full((1, D)), _full((1, D)), _full((D, 2 * D)), _full((1, 2 * D)),
                  _full((2 * D, D)), _full((1, D)), _full((1, D)), _full((1, D))],
        out_specs=_rows(D),
        out_shape=jax.ShapeDtypeStruct((N, D), jnp.float32),
    )(outm, outm, outz, outz, h, wo, bo, g1, b1n, w1, b1f, w2, b2f, g2, b2n)


def _read_body(h_ref, w0, b0, w1, b1, w2, b2, o_ref):
    t = jax.nn.relu(jnp.dot(h_ref[...], w0[...], preferred_element_type=jnp.float32) + b0[...])
    t = jax.nn.relu(jnp.dot(t, w1[...], preferred_element_type=jnp.float32) + b1[...])
    o_ref[...] = jnp.dot(t, w2[...], preferred_element_type=jnp.float32) + b2[...]


def _read(h, w0, b0, w1, b1, w2, b2):
    return pl.pallas_call(
        _read_body,
        grid=(GRID,),
        in_specs=[_rows(D), _full((D, D // 2)), _full((1, D // 2)),
                  _full((D // 2, D // 4)), _full((1, D // 4)),
                  _full((D // 4, 1)), _full((1, 1))],
        out_specs=_rows(1),
        out_shape=jax.ShapeDtypeStruct((N, 1), jnp.float32),
    )(h, w0, b0, w1, b1, w2, b2)


_GDN = lax.GatherDimensionNumbers(
    offset_dims=(), collapsed_slice_dims=(0,), start_index_map=(0,))

@functools.cache
def _edge_call():
    mesh = plsc.VectorSubcoreMesh(core_axis_name="c", subcore_axis_name="s")
    part = jax.ShapeDtypeStruct((2, NPAD, D), jnp.float32)
    return functools.partial(
        pl.kernel,
        out_type=(part, part),
        mesh=mesh,
        scratch_types=[
        pltpu.VMEM((EB,), jnp.int32),
        pltpu.VMEM((EB,), jnp.int32),
        pltpu.VMEM((EB,), jnp.int32),
        pltpu.VMEM((EB, D), jnp.float32),
        pltpu.VMEM((EB, D), jnp.float32),
        pltpu.VMEM((EB, D), jnp.float32),
        pltpu.VMEM((EB, DH), jnp.float32),
        pltpu.VMEM_SHARED((NPAD, D), jnp.float32),
        pltpu.VMEM_SHARED((NPAD, DH), jnp.float32),
        pltpu.SemaphoreType.DMA,
        ],
        compiler_params=pltpu.CompilerParams(use_tc_tiling_on_sc=False),
    )(_edge_body)


def _edge_body(k_hbm, q_hbm, v_hbm, i0_hbm, i1g_hbm, i1s_hbm,
               outm_hbm, outz_hbm,
               gi0, gi1, si1, kb, qb, vb, zbt, acc_m, acc_z, sem):
    c = lax.axis_index("c")
    s = lax.axis_index("s")
    wid = c * 16 + s
    zero16 = jnp.zeros((16,), jnp.float32)
    dbase = s * RPT

    def _zb_loop(r, _):
        for col in range(D // 16):
            kb[r, pl.ds(col * 16, 16)] = zero16
        zbt[r, pl.ds(0, 16)] = zero16
        return 0
    lax.fori_loop(0, EB, _zb_loop, 0)

    def _clear_loop(t, _):
        pltpu.sync_copy(kb, acc_m.at[pl.ds(dbase + t * EB, EB)])
        pltpu.sync_copy(zbt, acc_z.at[pl.ds(dbase + t * EB, EB)])
        return 0
    lax.fori_loop(0, ZF, _clear_loop, 0)
    pltpu.sync_copy(kb.at[pl.ds(0, ZT)], acc_m.at[pl.ds(dbase + ZF * EB, ZT)])
    pltpu.sync_copy(zbt.at[pl.ds(0, ZT)], acc_z.at[pl.ds(dbase + ZF * EB, ZT)])
    plsc.subcore_barrier()

    base = wid * EPW
    lane = lax.iota(jnp.int32, 16)
    rot_idx = [lax.rem(lane + r, 16) for r in (8, 4, 2, 1)]

    def _rotsum(xv):
        for ridx in rot_idx:
            xv = xv + lax.gather(xv, ridx[:, None], _GDN, slice_sizes=(1,),
                                 mode=lax.GatherScatterMode.PROMISE_IN_BOUNDS)
        return xv

    def _batch(t, _):
        off = base + t * EB
        pltpu.sync_copy(i0_hbm.at[pl.ds(off, EB)], gi0)
        pltpu.sync_copy(i1g_hbm.at[pl.ds(off, EB)], gi1)
        pltpu.sync_copy(i1s_hbm.at[pl.ds(off, EB)], si1)

        cp_k = pltpu.async_copy(k_hbm.at[gi0], kb, sem)
        cp_q = pltpu.async_copy(q_hbm.at[gi1], qb, sem)
        cp_v = pltpu.async_copy(v_hbm.at[gi0], vb, sem)
        cp_k.wait()
        cp_q.wait()
        cp_v.wait()

        def _edge(e, _):
            zrow = jnp.zeros((16,), jnp.float32)
            for hh in range(H):
                kh = kb[e, pl.ds(hh * DH, DH)]
                qh = qb[e, pl.ds(hh * DH, DH)]
                sc = _rotsum(kh * qh) * INV_SCALE
                pvec = jnp.exp(jnp.clip(sc, -5.0, 5.0))
                kb[e, pl.ds(hh * DH, DH)] = vb[e, pl.ds(hh * DH, DH)] * pvec
                zrow = jnp.where(lane == hh, pvec, zrow)
            zbt[e, pl.ds(0, 16)] = zrow
            return 0
        lax.fori_loop(0, EB, _edge, 0)

        pltpu.sync_copy(kb, acc_m.at[si1], add=True)
        pltpu.sync_copy(zbt, acc_z.at[si1], add=True)
        return 0
    lax.fori_loop(0, NB, _batch, 0)

    plsc.subcore_barrier()
    pltpu.sync_copy(acc_m.at[pl.ds(dbase, RPT)], outm_hbm.at[c, pl.ds(dbase, RPT)])

    def _zrows(nrows, ob):
        def _zrow(n, _):
            zv = zbt[n, pl.ds(0, 16)]
            for hh in range(H):
                kb[n, pl.ds(hh * DH, DH)] = jnp.broadcast_to(zv[hh], (16,))
            return 0
        lax.fori_loop(0, nrows, _zrow, 0)

    def _zchunk(t, _):
        pltpu.sync_copy(acc_z.at[pl.ds(dbase + t * EB, EB)], zbt)
        _zrows(EB, None)
        pltpu.sync_copy(kb, outz_hbm.at[c, pl.ds(dbase + t * EB, EB)])
        return 0
    lax.fori_loop(0, ZF, _zchunk, 0)

    pltpu.sync_copy(acc_z.at[pl.ds(dbase + ZF * EB, ZT)], zbt.at[pl.ds(0, ZT)])
    _zrows(ZT, None)
    pltpu.sync_copy(kb.at[pl.ds(0, ZT)], outz_hbm.at[c, pl.ds(dbase + ZF * EB, ZT)])


def kernel(x, edge_index, emb, Wq, bq, Wk, bk, Wv, bv, WO, bO, ln1_g, ln1_b,
           W1, b1, W2, b2, ln2_g, ln2_b, r0_W, r0_b, r1_W, r1_b, r2_W, r2_b):
    ei0 = edge_index[0]
    ei1 = edge_index[1]
    pad = E_PAD - E
    i0 = jnp.concatenate([ei0, jnp.zeros((pad,), jnp.int32)])
    i1g = jnp.concatenate([ei1, jnp.zeros((pad,), jnp.int32)])
    i1s = jnp.concatenate([ei1, jnp.full((pad,), N, jnp.int32)])

    h = _embed(x.reshape(N, 1), emb)
    for l in range(NL):
        q, k, v = _qkv(h, Wq[l], bq[l].reshape(1, D), Wk[l], bk[l].reshape(1, D),
                       Wv[l], bv[l].reshape(1, D))
        outm, outz = _edge_call()(k, q, v, i0, i1g, i1s)
        h = _post(outm, outz, h, WO[l], bO[l].reshape(1, D),
                  ln1_g[l].reshape(1, D), ln1_b[l].reshape(1, D),
                  W1[l], b1[l].reshape(1, 2 * D), W2[l], b2[l].reshape(1, D),
                  ln2_g[l].reshape(1, D), ln2_b[l].reshape(1, D))
    y = _read(h, r0_W, r0_b.reshape(1, D // 2), r1_W, r1_b.reshape(1, D // 4),
              r2_W, r2_b.reshape(1, 1))
    return y.reshape(1, N, 1)


# speedup vs baseline: 51.3236x; 1.6405x over previous
import functools

import jax
import jax.numpy as jnp
from jax import lax
from jax.experimental import pallas as pl
from jax.experimental.pallas import tpu as pltpu
from jax.experimental.pallas import tpu_sc as plsc

N = 10000
E = 160000
D = 128
H = 8
DH = 16
NL = 4
INV_SCALE = 1.0 / 4.0

NW = 32
EB = 48
NB = 106
EPW = EB * NB
E_PAD = NW * EPW
E_ALLOC = E_PAD + 2 * EB
NPAD = 10112
RPT = NPAD // 16
ZF = RPT // EB
ZT = RPT - ZF * EB

BN = 1000
GRID = N // BN


def _full(shape):
    return pl.BlockSpec(shape, lambda i: (0, 0))


def _rows(cols):
    return pl.BlockSpec((BN, cols), lambda i: (i, 0))


def _embed_body(x_ref, emb_ref, o_ref):
    ids = x_ref[...]
    iota = lax.broadcasted_iota(jnp.int32, (BN, 64), 1)
    onehot = (ids == iota).astype(jnp.float32)
    o_ref[...] = jnp.dot(onehot, emb_ref[...], preferred_element_type=jnp.float32)


def _embed(xr, emb):
    return pl.pallas_call(
        _embed_body,
        grid=(GRID,),
        in_specs=[pl.BlockSpec((BN, 1), lambda i: (i, 0)), _full((64, D))],
        out_specs=_rows(D),
        out_shape=jax.ShapeDtypeStruct((N, D), jnp.float32),
    )(xr, emb)


def _qkv_body(h_ref, wq, bq, wk, bk, wv, bv, q_o, k_o, v_o):
    hb = h_ref[...]
    q_o[...] = jnp.dot(hb, wq[...], preferred_element_type=jnp.float32) + bq[...]
    k_o[...] = jnp.dot(hb, wk[...], preferred_element_type=jnp.float32) + bk[...]
    v_o[...] = jnp.dot(hb, wv[...], preferred_element_type=jnp.float32) + bv[...]


def _qkv(h, wq, bq, wk, bk, wv, bv):
    out = jax.ShapeDtypeStruct((N, D), jnp.float32)
    return pl.pallas_call(
        _qkv_body,
        grid=(GRID,),
        in_specs=[_rows(D), _full((D, D)), _full((1, D)), _full((D, D)),
                  _full((1, D)), _full((D, D)), _full((1, D))],
        out_specs=(_rows(D), _rows(D), _rows(D)),
        out_shape=(out, out, out),
    )(h, wq, bq, wk, bk, wv, bv)


def _ln(xv, g, b):
    mu = jnp.mean(xv, axis=-1, keepdims=True)
    var = jnp.mean((xv - mu) ** 2, axis=-1, keepdims=True)
    return (xv - mu) * lax.rsqrt(var + 1e-5) * g + b


def _post_body(m0, m1, zr0, zr1, h_ref, wo, bo, g1, b1n, w1, b1f, w2, b2f, g2, b2n, o_ref):
    wv = m0[0] + m1[0]
    zrep = zr0[0] + zr1[0]
    att = wv / (zrep + 1e-6)
    y = jnp.dot(att, wo[...], preferred_element_type=jnp.float32) + bo[...]
    hb = h_ref[...]
    h1 = _ln(hb + y, g1[...], b1n[...])
    t = jax.nn.relu(jnp.dot(y, w1[...], preferred_element_type=jnp.float32) + b1f[...])
    y2 = jnp.dot(t, w2[...], preferred_element_type=jnp.float32) + b2f[...]
    o_ref[...] = _ln(h1 + y2, g2[...], b2n[...])


def _post(outm, outz, h, wo, bo, g1, b1n, w1, b1f, w2, b2f, g2, b2n):
    part0 = pl.BlockSpec((1, BN, D), lambda i: (0, i, 0))
    part1 = pl.BlockSpec((1, BN, D), lambda i: (1, i, 0))
    return pl.pallas_call(
        _post_body,
        grid=(GRID,),
        in_specs=[part0, part1, part0, part1, _rows(D), _full((D, D)), _full((1, D)),
                  _full((1, D)), _full((1, D)), _full((D, 2 * D)), _full((1, 2 * D)),
                  _full((2 * D, D)), _full((1, D)), _full((1, D)), _full((1, D))],
        out_specs=_rows(D),
        out_shape=jax.ShapeDtypeStruct((N, D), jnp.float32),
    )(outm, outm, outz, outz, h, wo, bo, g1, b1n, w1, b1f, w2, b2f, g2, b2n)


def _read_body(h_ref, w0, b0, w1, b1, w2, b2, o_ref):
    t = jax.nn.relu(jnp.dot(h_ref[...], w0[...], preferred_element_type=jnp.float32) + b0[...])
    t = jax.nn.relu(jnp.dot(t, w1[...], preferred_element_type=jnp.float32) + b1[...])
    o_ref[...] = jnp.dot(t, w2[...], preferred_element_type=jnp.float32) + b2[...]


def _read(h, w0, b0, w1, b1, w2, b2):
    return pl.pallas_call(
        _read_body,
        grid=(GRID,),
        in_specs=[_rows(D), _full((D, D // 2)), _full((1, D // 2)),
                  _full((D // 2, D // 4)), _full((1, D // 4)),
                  _full((D // 4, 1)), _full((1, 1))],
        out_specs=_rows(1),
        out_shape=jax.ShapeDtypeStruct((N, 1), jnp.float32),
    )(h, w0, b0, w1, b1, w2, b2)


_GDN = lax.GatherDimensionNumbers(
    offset_dims=(), collapsed_slice_dims=(0,), start_index_map=(0,))

@functools.cache
def _edge_call():
    mesh = plsc.VectorSubcoreMesh(core_axis_name="c", subcore_axis_name="s")
    part = jax.ShapeDtypeStruct((2, NPAD, D), jnp.float32)
    return functools.partial(
        pl.kernel,
        out_type=(part, part),
        mesh=mesh,
        scratch_types=[
        pltpu.VMEM((EB,), jnp.int32),
        pltpu.VMEM((EB,), jnp.int32),
        pltpu.VMEM((EB,), jnp.int32),
        pltpu.VMEM((EB,), jnp.int32),
        pltpu.VMEM((EB,), jnp.int32),
        pltpu.VMEM((EB,), jnp.int32),
        pltpu.VMEM((EB, D), jnp.float32),
        pltpu.VMEM((EB, D), jnp.float32),
        pltpu.VMEM((EB, D), jnp.float32),
        pltpu.VMEM((EB, D), jnp.float32),
        pltpu.VMEM((EB, D), jnp.float32),
        pltpu.VMEM((EB, D), jnp.float32),
        pltpu.VMEM((EB, DH), jnp.float32),
        pltpu.VMEM((EB, DH), jnp.float32),
        pltpu.VMEM_SHARED((NPAD, D), jnp.float32),
        pltpu.VMEM_SHARED((NPAD, DH), jnp.float32),
        pltpu.SemaphoreType.DMA,
        pltpu.SemaphoreType.DMA,
        pltpu.SemaphoreType.DMA,
        ],
        compiler_params=pltpu.CompilerParams(use_tc_tiling_on_sc=False),
    )(_edge_body)


def _edge_body(k_hbm, q_hbm, v_hbm, i0_hbm, i1g_hbm, i1s_hbm,
               outm_hbm, outz_hbm,
               gi0a, gi1a, si1a, gi0b, gi1b, si1b,
               kba, qba, vba, kbb, qbb, vbb, zbta, zbtb,
               acc_m, acc_z, sem_i, sem_ga, sem_gb):
    c = lax.axis_index("c")
    s = lax.axis_index("s")
    wid = c * 16 + s
    zero16 = jnp.zeros((16,), jnp.float32)
    dbase = s * RPT
    kb, qb, vb, zbt = kba, qba, vba, zbta

    def _zb_loop(r, _):
        for col in range(D // 16):
            kb[r, pl.ds(col * 16, 16)] = zero16
        zbt[r, pl.ds(0, 16)] = zero16
        return 0
    lax.fori_loop(0, EB, _zb_loop, 0)

    def _clear_loop(t, _):
        pltpu.sync_copy(kb, acc_m.at[pl.ds(dbase + t * EB, EB)])
        pltpu.sync_copy(zbt, acc_z.at[pl.ds(dbase + t * EB, EB)])
        return 0
    lax.fori_loop(0, ZF, _clear_loop, 0)
    pltpu.sync_copy(kb.at[pl.ds(0, ZT)], acc_m.at[pl.ds(dbase + ZF * EB, ZT)])
    pltpu.sync_copy(zbt.at[pl.ds(0, ZT)], acc_z.at[pl.ds(dbase + ZF * EB, ZT)])
    plsc.subcore_barrier()

    base = wid * EPW
    lane = lax.iota(jnp.int32, 16)
    rot_idx = [lax.rem(lane + r, 16) for r in (8, 4, 2, 1)]

    def _rotsum(xv):
        for ridx in rot_idx:
            xv = xv + lax.gather(xv, ridx[:, None], _GDN, slice_sizes=(1,),
                                 mode=lax.GatherScatterMode.PROMISE_IN_BOUNDS)
        return xv

    def _idx_start(off, g0, g1, s1):
        pltpu.async_copy(i0_hbm.at[off], g0, sem_i)
        pltpu.async_copy(i1g_hbm.at[off], g1, sem_i)
        pltpu.async_copy(i1s_hbm.at[off], s1, sem_i)

    def _idx_wait(g0, g1, s1):
        pltpu.make_async_copy(i0_hbm.at[pl.ds(0, EB)], g0, sem_i).wait()
        pltpu.make_async_copy(i1g_hbm.at[pl.ds(0, EB)], g1, sem_i).wait()
        pltpu.make_async_copy(i1s_hbm.at[pl.ds(0, EB)], s1, sem_i).wait()

    def _gat_start(g0, g1, kd, qd, vd, sem):
        pltpu.async_copy(k_hbm.at[g0], kd, sem)
        pltpu.async_copy(q_hbm.at[g1], qd, sem)
        pltpu.async_copy(v_hbm.at[g0], vd, sem)

    def _gat_wait(g0, g1, kd, qd, vd, sem):
        pltpu.make_async_copy(k_hbm.at[g0], kd, sem).wait()
        pltpu.make_async_copy(q_hbm.at[g1], qd, sem).wait()
        pltpu.make_async_copy(v_hbm.at[g0], vd, sem).wait()

    def _compute(kd, qd, vd, zd, s1):
        def _edge(e, _):
            zrow = jnp.zeros((16,), jnp.float32)
            for hh in range(H):
                kh = kd[e, pl.ds(hh * DH, DH)]
                qh = qd[e, pl.ds(hh * DH, DH)]
                sc = _rotsum(kh * qh) * INV_SCALE
                pvec = jnp.exp(jnp.clip(sc, -5.0, 5.0))
                kd[e, pl.ds(hh * DH, DH)] = vd[e, pl.ds(hh * DH, DH)] * pvec
                zrow = jnp.where(lane == hh, pvec, zrow)
            zd[e, pl.ds(0, 16)] = zrow
            return 0
        lax.fori_loop(0, EB, _edge, 0)
        pltpu.sync_copy(kd, acc_m.at[s1], add=True)
        pltpu.sync_copy(zd, acc_z.at[s1], add=True)

    pltpu.sync_copy(i0_hbm.at[pl.ds(base, EB)], gi0a)
    pltpu.sync_copy(i1g_hbm.at[pl.ds(base, EB)], gi1a)
    pltpu.sync_copy(i1s_hbm.at[pl.ds(base, EB)], si1a)
    _gat_start(gi0a, gi1a, kba, qba, vba, sem_ga)
    _idx_start(pl.ds(base + EB, EB), gi0b, gi1b, si1b)

    def _pair(tt, _):
        t0 = 2 * tt
        _idx_wait(gi0b, gi1b, si1b)
        _gat_start(gi0b, gi1b, kbb, qbb, vbb, sem_gb)
        _gat_wait(gi0a, gi1a, kba, qba, vba, sem_ga)
        _compute(kba, qba, vba, zbta, si1a)
        _idx_start(pl.ds(base + (t0 + 2) * EB, EB), gi0a, gi1a, si1a)
        _idx_wait(gi0a, gi1a, si1a)
        _gat_start(gi0a, gi1a, kba, qba, vba, sem_ga)
        _gat_wait(gi0b, gi1b, kbb, qbb, vbb, sem_gb)
        _compute(kbb, qbb, vbb, zbtb, si1b)
        _idx_start(pl.ds(base + (t0 + 3) * EB, EB), gi0b, gi1b, si1b)
        return 0
    lax.fori_loop(0, NB // 2, _pair, 0)

    _gat_wait(gi0a, gi1a, kba, qba, vba, sem_ga)
    _idx_wait(gi0b, gi1b, si1b)

    plsc.subcore_barrier()
    pltpu.sync_copy(acc_m.at[pl.ds(dbase, RPT)], outm_hbm.at[c, pl.ds(dbase, RPT)])

    def _zrows(nrows, ob):
        def _zrow(n, _):
            zv = zbt[n, pl.ds(0, 16)]
            for hh in range(H):
                kb[n, pl.ds(hh * DH, DH)] = jnp.broadcast_to(zv[hh], (16,))
            return 0
        lax.fori_loop(0, nrows, _zrow, 0)

    def _zchunk(t, _):
        pltpu.sync_copy(acc_z.at[pl.ds(dbase + t * EB, EB)], zbt)
        _zrows(EB, None)
        pltpu.sync_copy(kb, outz_hbm.at[c, pl.ds(dbase + t * EB, EB)])
        return 0
    lax.fori_loop(0, ZF, _zchunk, 0)

    pltpu.sync_copy(acc_z.at[pl.ds(dbase + ZF * EB, ZT)], zbt.at[pl.ds(0, ZT)])
    _zrows(ZT, None)
    pltpu.sync_copy(kb.at[pl.ds(0, ZT)], outz_hbm.at[c, pl.ds(dbase + ZF * EB, ZT)])


def kernel(x, edge_index, emb, Wq, bq, Wk, bk, Wv, bv, WO, bO, ln1_g, ln1_b,
           W1, b1, W2, b2, ln2_g, ln2_b, r0_W, r0_b, r1_W, r1_b, r2_W, r2_b):
    ei0 = edge_index[0]
    ei1 = edge_index[1]
    pad = E_ALLOC - E
    i0 = jnp.concatenate([ei0, jnp.zeros((pad,), jnp.int32)])
    i1g = jnp.concatenate([ei1, jnp.zeros((pad,), jnp.int32)])
    i1s = jnp.concatenate([ei1, jnp.full((pad,), N, jnp.int32)])

    h = _embed(x.reshape(N, 1), emb)
    for l in range(NL):
        q, k, v = _qkv(h, Wq[l], bq[l].reshape(1, D), Wk[l], bk[l].reshape(1, D),
                       Wv[l], bv[l].reshape(1, D))
        outm, outz = _edge_call()(k, q, v, i0, i1g, i1s)
        h = _post(outm, outz, h, WO[l], bO[l].reshape(1, D),
                  ln1_g[l].reshape(1, D), ln1_b[l].reshape(1, D),
                  W1[l], b1[l].reshape(1, 2 * D), W2[l], b2[l].reshape(1, D),
                  ln2_g[l].reshape(1, D), ln2_b[l].reshape(1, D))
    y = _read(h, r0_W, r0_b.reshape(1, D // 2), r1_W, r1_b.reshape(1, D // 4),
              r2_W, r2_b.reshape(1, 1))
    return y.reshape(1, N, 1)
